# baseline (speedup 1.0000x reference)
import functools

import jax
import jax.numpy as jnp
from jax import lax
from jax.experimental import pallas as pl
from jax.experimental.pallas import tpu as pltpu
from jax.experimental.pallas import tpu_sc as plsc

N = 100000
E = 6400000

NC = 2
NS = 16
NW = NC * NS
L = 16

CHUNK_E = 2048
CHUNKS = E // CHUNK_E

NSLICE = 6400
N_PAD = NS * NSLICE
SUB = N_PAD // 128

_mesh = plsc.VectorSubcoreMesh(
    core_axis_name="c", subcore_axis_name="s", num_cores=NC, num_subcores=NS)
_sc_params = pltpu.CompilerParams(needs_layout_passes=False)

_f32 = jnp.float32
_out1 = jax.ShapeDtypeStruct((NC * N_PAD,), _f32)


def _fill(ref, n, value):
  v = jnp.full((L,), value, _f32)

  @pl.loop(0, n // L, unroll=8)
  def _(i):
    ref[pl.ds(i * L, L)] = v


def _frsqrt(d):
  ii = plsc.bitcast(d, jnp.int32)
  ii = 0x5F3759DF - lax.shift_right_logical(ii, 1)
  y = plsc.bitcast(ii, _f32)
  h = 0.5 * d
  for _ in range(3):
    y = y * (1.5 - h * y * y)
  return y


def _edge_pass_body(mode, *refs):
  if mode == "deg":
    (dst_hbm, out_hbm, idx_d, val_v, zbuf, acc_sh, dma_sem, sc_sem) = refs
  elif mode == "l1":
    (src_hbm, dst_hbm, x_hbm, pdeg_hbm, out_hbm, dis_hbm, u_hbm,
     table_v, idx_s, idx_d, val_v, zbuf, acc_sh, dma_sem, sc_sem) = refs
  else:
    (src_hbm, dst_hbm, p1_hbm, dis_hbm, u1_hbm, w_hbm, out_hbm, u_hbm,
     table_v, idx_s, idx_d, val_v, zbuf, acc_sh, dma_sem, sc_sem, wbuf) = refs

  c = lax.axis_index("c")
  s = lax.axis_index("s")
  w = c * NS + s

  base = CHUNKS // NW
  rem = CHUNKS % NW
  nch = base + jnp.where(w < rem, 1, 0)
  ch0 = w * base + jnp.minimum(w, rem)

  def bslice(ref, b):
    return ref.at[pl.ds(pl.multiple_of(b * CHUNK_E, CHUNK_E), CHUNK_E)]

  def start_chunk(ci, b):
    off = pl.multiple_of((ch0 + ci) * CHUNK_E, CHUNK_E)
    if mode != "deg":
      pltpu.async_copy(src_hbm.at[pl.ds(off, CHUNK_E)], bslice(idx_s, b),
                       dma_sem.at[b])
    pltpu.async_copy(dst_hbm.at[pl.ds(off, CHUNK_E)], bslice(idx_d, b),
                     dma_sem.at[b])

  def wait_chunk(ci, b):
    off = pl.multiple_of((ch0 + ci) * CHUNK_E, CHUNK_E)
    if mode != "deg":
      pltpu.make_async_copy(src_hbm.at[pl.ds(off, CHUNK_E)], bslice(idx_s, b),
                            dma_sem.at[b]).wait()
    pltpu.make_async_copy(dst_hbm.at[pl.ds(off, CHUNK_E)], bslice(idx_d, b),
                          dma_sem.at[b]).wait()

  def fire_scatters(b):
    vb = b if mode != "deg" else 0
    pltpu.async_copy(bslice(val_v, vb), acc_sh.at[bslice(idx_d, b)],
                     sc_sem.at[b], add=True)

  def drain_scatters(b):
    vb = b if mode != "deg" else 0
    pltpu.make_async_copy(bslice(val_v, vb), acc_sh.at[bslice(idx_d, b)],
                          sc_sem.at[b]).wait()

  start_chunk(0, 0)

  noff = s * NSLICE
  coff = c * N_PAD + noff

  _fill(zbuf, NSLICE, 0.0)
  pltpu.sync_copy(zbuf, acc_sh.at[pl.ds(noff, NSLICE)])

  if mode == "deg":
    _fill(val_v, CHUNK_E, 1.0)
  elif mode == "l1":
    pltpu.sync_copy(pdeg_hbm.at[pl.ds(noff, NSLICE)],
                    table_v.at[pl.ds(0, NSLICE)])
    pltpu.sync_copy(pdeg_hbm.at[pl.ds(N_PAD + noff, NSLICE)],
                    table_v.at[pl.ds(NSLICE, NSLICE)])
    pltpu.sync_copy(x_hbm.at[pl.ds(noff, NSLICE)],
                    table_v.at[pl.ds(2 * NSLICE, NSLICE)])

    @plsc.parallel_loop(0, NSLICE // L, unroll=4)
    def _(j):
      o = j * L
      deg = table_v[pl.ds(o, L)] + table_v[pl.ds(NSLICE + o, L)] + 1.0
      dis = _frsqrt(deg)
      u1 = table_v[pl.ds(2 * NSLICE + o, L)] * dis
      zbuf[pl.ds(o, L)] = dis
      table_v[pl.ds(3 * NSLICE + o, L)] = u1

    pltpu.sync_copy(zbuf, dis_hbm.at[pl.ds(coff, NSLICE)])
    pltpu.sync_copy(table_v.at[pl.ds(3 * NSLICE, NSLICE)],
                    u_hbm.at[pl.ds(coff, NSLICE)])
  else:
    pltpu.sync_copy(p1_hbm.at[pl.ds(noff, NSLICE)],
                    table_v.at[pl.ds(0, NSLICE)])
    pltpu.sync_copy(p1_hbm.at[pl.ds(N_PAD + noff, NSLICE)],
                    table_v.at[pl.ds(NSLICE, NSLICE)])
    pltpu.sync_copy(u1_hbm.at[pl.ds(coff, NSLICE)],
                    table_v.at[pl.ds(2 * NSLICE, NSLICE)])
    pltpu.sync_copy(dis_hbm.at[pl.ds(coff, NSLICE)], zbuf)
    pltpu.sync_copy(w_hbm, wbuf)

    w1v = wbuf[pl.ds(0, L)]
    b1v = wbuf[pl.ds(L, L)]
    w2v = wbuf[pl.ds(2 * L, L)]

    @plsc.parallel_loop(0, NSLICE // L, unroll=2)
    def _(j):
      o = j * L
      dis = zbuf[pl.ds(o, L)]
      a1 = dis * (table_v[pl.ds(o, L)] + table_v[pl.ds(NSLICE + o, L)]
                  + table_v[pl.ds(2 * NSLICE + o, L)])
      z = jnp.zeros((L,), _f32)
      for k in range(16):
        z = z + jnp.maximum(a1 * w1v[k] + b1v[k], 0.0) * w2v[k]
      table_v[pl.ds(3 * NSLICE + o, L)] = z * dis

    pltpu.sync_copy(table_v.at[pl.ds(3 * NSLICE, NSLICE)],
                    u_hbm.at[pl.ds(coff, NSLICE)])

  plsc.subcore_barrier()

  if mode != "deg":
    pltpu.sync_copy(u_hbm.at[pl.ds(c * N_PAD, N_PAD)], table_v)

  @pl.loop(0, nch)
  def _(ci):
    b = ci % 2
    nb = 1 - b

    @pl.when(ci > 0)
    def _():
      drain_scatters(nb)

    @pl.when(ci + 1 < nch)
    def _():
      start_chunk(ci + 1, nb)

    wait_chunk(ci, b)

    if mode != "deg":
      boff = b * CHUNK_E

      @plsc.parallel_loop(0, CHUNK_E // L, unroll=8)
      def _(g):
        sidx = idx_s[pl.ds(boff + g * L, L)]
        val_v[pl.ds(boff + g * L, L)] = plsc.load_gather(table_v, [sidx])

    fire_scatters(b)

  drain_scatters((nch - 1) % 2)

  plsc.subcore_barrier()
  pltpu.sync_copy(acc_sh.at[pl.ds(noff, NSLICE)],
                  out_hbm.at[pl.ds(c * N_PAD + noff, NSLICE)])


_scatter_scratch = [
    pltpu.VMEM((2 * CHUNK_E,), jnp.int32),
    pltpu.VMEM((2 * CHUNK_E,), _f32),
    pltpu.VMEM((NSLICE,), _f32),
    pltpu.VMEM_SHARED((N_PAD,), _f32),
    pltpu.SemaphoreType.DMA((2,)),
    pltpu.SemaphoreType.DMA((2,)),
]

_gather_scratch = [pltpu.VMEM((N_PAD,), _f32),
                   pltpu.VMEM((2 * CHUNK_E,), jnp.int32)] + _scatter_scratch

_deg_pass = pl.kernel(
    functools.partial(_edge_pass_body, "deg"),
    out_type=_out1,
    mesh=_mesh,
    scratch_types=_scatter_scratch,
    compiler_params=_sc_params,
    name="sc_degree_pass",
)

_l1_pass = pl.kernel(
    functools.partial(_edge_pass_body, "l1"),
    out_type=(_out1, _out1, _out1),
    mesh=_mesh,
    scratch_types=_gather_scratch,
    compiler_params=_sc_params,
    name="sc_layer1_pass",
)

_l2_pass = pl.kernel(
    functools.partial(_edge_pass_body, "l2"),
    out_type=(_out1, _out1),
    mesh=_mesh,
    scratch_types=_gather_scratch + [pltpu.VMEM((48,), _f32)],
    compiler_params=_sc_params,
    name="sc_layer2_pass",
)


def _tc_out_body(p2_ref, dis_ref, u2_ref, b2_ref, out_ref):
  out_ref[...] = dis_ref[0] * (p2_ref[0] + p2_ref[1] + u2_ref[0]) + b2_ref[0, 0]


_vmem_spec = pl.BlockSpec(memory_space=pltpu.VMEM)
_smem_spec = pl.BlockSpec(memory_space=pltpu.SMEM)

_tc_out = pl.pallas_call(
    _tc_out_body,
    in_specs=[_vmem_spec, _vmem_spec, _vmem_spec, _smem_spec],
    out_specs=_vmem_spec,
    out_shape=jax.ShapeDtypeStruct((SUB, 128), _f32),
)


@jax.jit
def kernel(x, edge_index, W1, b1, W2, b2):
  src1d = edge_index[0]
  dst1d = edge_index[1]

  xp = jnp.pad(x[:, 0], (0, N_PAD - N))
  wpack = jnp.concatenate([W1.reshape(16), b1.reshape(16), W2.reshape(16)])

  pdeg = _deg_pass(dst1d)
  p1, dis, u1 = _l1_pass(src1d, dst1d, xp, pdeg)
  p2, u2 = _l2_pass(src1d, dst1d, p1, dis, u1, wpack)

  out = _tc_out(p2.reshape(NC, SUB, 128), dis.reshape(NC, SUB, 128),
                u2.reshape(NC, SUB, 128), b2.reshape(1, 1))

  return out.reshape(N_PAD)[:N].reshape(N, 1)


# speedup vs baseline: 1.0453x; 1.0453x over previous
import functools

import jax
import jax.numpy as jnp
from jax import lax
from jax.experimental import pallas as pl
from jax.experimental.pallas import tpu as pltpu
from jax.experimental.pallas import tpu_sc as plsc

N = 100000
E = 6400000

NC = 2
NS = 16
NW = NC * NS
L = 16

CHUNK_E = 2560
CHUNKS = E // CHUNK_E

NSLICE = 6400
N_PAD = NS * NSLICE
SUB = N_PAD // 128

_mesh = plsc.VectorSubcoreMesh(
    core_axis_name="c", subcore_axis_name="s", num_cores=NC, num_subcores=NS)
_sc_params = pltpu.CompilerParams(needs_layout_passes=False)


def _fill(ref, n, value):
  v = jnp.full((L,), value, jnp.float32)

  @pl.loop(0, n // L, unroll=8)
  def _(i):
    ref[pl.ds(i * L, L)] = v


def _edge_pass_body(with_gather, *refs):
  if with_gather:
    (src_hbm, dst_hbm, table_hbm, out_hbm,
     table_v, idx_s, idx_d, val_v, zbuf, acc_sh, dma_sem, sc_sem) = refs
  else:
    (dst_hbm, out_hbm, idx_d, val_v, zbuf, acc_sh, dma_sem, sc_sem) = refs

  c = lax.axis_index("c")
  s = lax.axis_index("s")
  w = c * NS + s

  base = CHUNKS // NW
  rem = CHUNKS % NW
  nch = base + jnp.where(w < rem, 1, 0)
  ch0 = w * base + jnp.minimum(w, rem)

  def bslice(ref, b):
    return ref.at[pl.ds(pl.multiple_of(b * CHUNK_E, CHUNK_E), CHUNK_E)]

  def start_chunk(ci, b):
    off = pl.multiple_of((ch0 + ci) * CHUNK_E, CHUNK_E)
    if with_gather:
      pltpu.async_copy(src_hbm.at[pl.ds(off, CHUNK_E)], bslice(idx_s, b),
                       dma_sem.at[b])
    pltpu.async_copy(dst_hbm.at[pl.ds(off, CHUNK_E)], bslice(idx_d, b),
                     dma_sem.at[b])

  def wait_chunk(ci, b):
    off = pl.multiple_of((ch0 + ci) * CHUNK_E, CHUNK_E)
    if with_gather:
      pltpu.make_async_copy(src_hbm.at[pl.ds(off, CHUNK_E)], bslice(idx_s, b),
                            dma_sem.at[b]).wait()
    pltpu.make_async_copy(dst_hbm.at[pl.ds(off, CHUNK_E)], bslice(idx_d, b),
                          dma_sem.at[b]).wait()

  def fire_scatters(b):
    vb = b if with_gather else 0
    pltpu.async_copy(bslice(val_v, vb), acc_sh.at[bslice(idx_d, b)],
                     sc_sem.at[b], add=True)

  def drain_scatters(b):
    vb = b if with_gather else 0
    pltpu.make_async_copy(bslice(val_v, vb), acc_sh.at[bslice(idx_d, b)],
                          sc_sem.at[b]).wait()

  _fill(zbuf, NSLICE // 2, 0.0)
  pltpu.sync_copy(zbuf, acc_sh.at[pl.ds(s * NSLICE, NSLICE // 2)])
  pltpu.sync_copy(zbuf, acc_sh.at[pl.ds(s * NSLICE + NSLICE // 2, NSLICE // 2)])

  start_chunk(0, 0)

  if with_gather:
    pltpu.sync_copy(table_hbm, table_v)
  else:
    _fill(val_v, CHUNK_E, 1.0)

  plsc.subcore_barrier()

  @pl.loop(0, nch)
  def _(ci):
    b = ci % 2
    nb = 1 - b

    @pl.when(ci > 0)
    def _():
      drain_scatters(nb)

    @pl.when(ci + 1 < nch)
    def _():
      start_chunk(ci + 1, nb)

    wait_chunk(ci, b)

    if with_gather:
      boff = b * CHUNK_E

      @plsc.parallel_loop(0, CHUNK_E // L, unroll=8)
      def _(g):
        sidx = idx_s[pl.ds(boff + g * L, L)]
        val_v[pl.ds(boff + g * L, L)] = plsc.load_gather(table_v, [sidx])

    fire_scatters(b)

  drain_scatters((nch - 1) % 2)

  plsc.subcore_barrier()
  pltpu.sync_copy(acc_sh.at[pl.ds(s * NSLICE, NSLICE)],
                  out_hbm.at[pl.ds(c * N_PAD + s * NSLICE, NSLICE)])


_scatter_scratch = [
    pltpu.VMEM((2 * CHUNK_E,), jnp.int32),
    pltpu.VMEM((2 * CHUNK_E,), jnp.float32),
    pltpu.VMEM((NSLICE // 2,), jnp.float32),
    pltpu.VMEM_SHARED((N_PAD,), jnp.float32),
    pltpu.SemaphoreType.DMA((2,)),
    pltpu.SemaphoreType.DMA((2,)),
]

_deg_pass = pl.kernel(
    functools.partial(_edge_pass_body, False),
    out_type=jax.ShapeDtypeStruct((NC * N_PAD,), jnp.float32),
    mesh=_mesh,
    scratch_types=_scatter_scratch,
    compiler_params=_sc_params,
    name="sc_degree_pass",
)

_agg_pass = pl.kernel(
    functools.partial(_edge_pass_body, True),
    out_type=jax.ShapeDtypeStruct((NC * N_PAD,), jnp.float32),
    mesh=_mesh,
    scratch_types=[pltpu.VMEM((N_PAD,), jnp.float32),
                   pltpu.VMEM((2 * CHUNK_E,), jnp.int32)] + _scatter_scratch,
    compiler_params=_sc_params,
    name="sc_aggregate_pass",
)


def _tc_norm_body(pdeg_ref, xp_ref, dis_ref, u1_ref):
  deg = pdeg_ref[0] + pdeg_ref[1] + 1.0
  dis = lax.rsqrt(deg)
  dis_ref[...] = dis
  u1_ref[...] = xp_ref[...] * dis


def _tc_mid_body(p1_ref, dis_ref, u1_ref, w1_ref, b1_ref, w2_ref, u2_ref):
  dis = dis_ref[...]
  a1 = dis * (p1_ref[0] + p1_ref[1] + u1_ref[...])
  z = jnp.zeros_like(a1)
  for k in range(16):
    z = z + jnp.maximum(a1 * w1_ref[0, k] + b1_ref[0, k], 0.0) * w2_ref[0, k]
  u2_ref[...] = z * dis


def _tc_out_body(p2_ref, dis_ref, u2_ref, b2_ref, out_ref):
  out_ref[...] = dis_ref[...] * (p2_ref[0] + p2_ref[1] + u2_ref[...]) + b2_ref[0, 0]


_vmem_spec = pl.BlockSpec(memory_space=pltpu.VMEM)
_smem_spec = pl.BlockSpec(memory_space=pltpu.SMEM)
_nd = jax.ShapeDtypeStruct((SUB, 128), jnp.float32)

_tc_norm = pl.pallas_call(
    _tc_norm_body,
    in_specs=[_vmem_spec, _vmem_spec],
    out_specs=[_vmem_spec, _vmem_spec],
    out_shape=[_nd, _nd],
)

_tc_mid = pl.pallas_call(
    _tc_mid_body,
    in_specs=[_vmem_spec, _vmem_spec, _vmem_spec,
              _smem_spec, _smem_spec, _smem_spec],
    out_specs=_vmem_spec,
    out_shape=_nd,
)

_tc_out = pl.pallas_call(
    _tc_out_body,
    in_specs=[_vmem_spec, _vmem_spec, _vmem_spec, _smem_spec],
    out_specs=_vmem_spec,
    out_shape=_nd,
)


@jax.jit
def kernel(x, edge_index, W1, b1, W2, b2):
  src1d = edge_index[0]
  dst1d = edge_index[1]

  xp = jnp.pad(x[:, 0], (0, N_PAD - N)).reshape(SUB, 128)

  pdeg = _deg_pass(dst1d).reshape(NC, SUB, 128)
  dis, u1 = _tc_norm(pdeg, xp)

  p1 = _agg_pass(src1d, dst1d, u1.reshape(N_PAD)).reshape(NC, SUB, 128)
  u2 = _tc_mid(p1, dis, u1,
               W1.reshape(1, 16), b1.reshape(1, 16), W2.reshape(1, 16))

  p2 = _agg_pass(src1d, dst1d, u2.reshape(N_PAD)).reshape(NC, SUB, 128)
  out = _tc_out(p2, dis, u2, b2.reshape(1, 1))

  return out.reshape(N_PAD)[:N].reshape(N, 1)
